# baseline (speedup 1.0000x reference)
import functools

import jax
import jax.numpy as jnp
from jax.experimental import pallas as pl
from jax.experimental.pallas import tpu as pltpu

N = 8192
F = 50
H = 20
G3 = 3 * H
BLK = 256
FA = F + 1

_G1W, _G2W, _G1B, _G2B, _WI0 = 0, F, 2 * F, 2 * F + 1, 2 * F + 2
_P50R = _WI0 + G3
_WH0, _WI1, _WH1, _H00, _H01 = 0, G3, 2 * G3, 3 * G3, 3 * G3 + 1
_P20R = _H01 + 1
_BI0, _BH0, _BI1, _BH1 = 0, 1, 2, 3
_P60R = 4


def _dot(a, b):
    return jnp.dot(a, b, preferred_element_type=jnp.float32)


def _dot_nt(a, b):
    return jax.lax.dot_general(a, b, (((1,), (1,)), ((), ())),
                               preferred_element_type=jnp.float32)


def _body(s_ref, p50_ref, p20_ref, p60_ref, emb_ref, adj_ref, adj8_ref,
          out_ref, ro_all_ref):
    j = pl.program_id(0)

    t = jax.lax.dot_general(
        adj_ref[...], emb_ref[...], (((1,), (0,)), ((), ())),
        preferred_element_type=jnp.float32)
    ro = jnp.maximum(
        _dot_nt(t[:, 0:F], p50_ref[_G1W:_G1W + F, :])
        + t[:, F:FA] * p50_ref[_G1B:_G1B + 1, :], 0.0)
    ro_all_ref[pl.ds(j * BLK, BLK), :] = ro

    @pl.when(j == pl.num_programs(0) - 1)
    def _fin():
        sub = s_ref[1]
        oh = (jax.lax.broadcasted_iota(jnp.int32, (1, 8), 1) == sub
              ).astype(jnp.float32)
        acc8 = _dot(adj8_ref[...], ro_all_ref[...])
        row = _dot(oh, acc8)
        ssum = _dot(oh, jnp.sum(adj8_ref[...], axis=1, keepdims=True))
        g = (_dot_nt(row, p50_ref[_G2W:_G2W + F, :])
             + ssum * p50_ref[_G2B:_G2B + 1, :])
        m = jnp.max(g, axis=1, keepdims=True)
        e = jnp.exp(g - m)
        v = g - m - jnp.log(jnp.sum(e, axis=1, keepdims=True))

        def gru(inp, h, wi, bi, wh, bh):
            gi = _dot_nt(inp, wi) + bi
            gh = _dot_nt(h, wh) + bh
            r = jax.nn.sigmoid(gi[:, 0:H] + gh[:, 0:H])
            z = jax.nn.sigmoid(gi[:, H:2 * H] + gh[:, H:2 * H])
            n = jnp.tanh(gi[:, 2 * H:G3] + r * gh[:, 2 * H:G3])
            return (1.0 - z) * n + z * h

        h0n = gru(v, p20_ref[_H00:_H00 + 1, :],
                  p50_ref[_WI0:_WI0 + G3, :], p60_ref[_BI0:_BI0 + 1, :],
                  p20_ref[_WH0:_WH0 + G3, :], p60_ref[_BH0:_BH0 + 1, :])
        h1n = gru(h0n, p20_ref[_H01:_H01 + 1, :],
                  p20_ref[_WI1:_WI1 + G3, :], p60_ref[_BI1:_BI1 + 1, :],
                  p20_ref[_WH1:_WH1 + G3, :], p60_ref[_BH1:_BH1 + 1, :])
        out_ref[...] = h1n


@functools.partial(jax.jit, static_argnames=())
def kernel(x, entity_emb, adj, gc1_w, gc1_b, gc2_w, gc2_b,
           w_ih0, w_hh0, b_ih0, b_hh0, w_ih1, w_hh1, b_ih1, b_hh1, h0):
    xi = jnp.asarray(x, jnp.int32)
    scalars = jnp.stack([xi // 8, xi % 8]).astype(jnp.int32)

    emb_aug = jnp.concatenate(
        [entity_emb, jnp.ones((N, 1), jnp.float32)], axis=1
    ).astype(jnp.bfloat16)

    p50 = jnp.concatenate(
        [gc1_w, gc2_w, gc1_b.reshape(1, F), gc2_b.reshape(1, F), w_ih0],
        axis=0)
    p20 = jnp.concatenate([w_hh0, w_ih1, w_hh1, h0[0], h0[1]], axis=0)
    p60 = jnp.concatenate(
        [b_ih0.reshape(1, G3), b_hh0.reshape(1, G3),
         b_ih1.reshape(1, G3), b_hh1.reshape(1, G3)], axis=0)

    G = N // BLK
    grid_spec = pltpu.PrefetchScalarGridSpec(
        num_scalar_prefetch=1,
        grid=(G,),
        in_specs=[
            pl.BlockSpec((_P50R, F), lambda j, s: (0, 0)),
            pl.BlockSpec((_P20R, H), lambda j, s: (0, 0)),
            pl.BlockSpec((_P60R, G3), lambda j, s: (0, 0)),
            pl.BlockSpec((N, FA), lambda j, s: (0, 0)),
            pl.BlockSpec((BLK, N), lambda j, s: (j, 0)),
            pl.BlockSpec((8, N), lambda j, s: (s[0], 0)),
        ],
        out_specs=pl.BlockSpec((1, H), lambda j, s: (0, 0)),
        scratch_shapes=[
            pltpu.VMEM((N, F), jnp.float32),
        ],
    )

    out = pl.pallas_call(
        _body,
        grid_spec=grid_spec,
        out_shape=jax.ShapeDtypeStruct((1, H), jnp.float32),
    )(scalars, p50, p20, p60, emb_aug, adj, adj)
    return out.reshape(-1)
